# Initial kernel scaffold; baseline (speedup 1.0000x reference)
import jax
import jax.numpy as jnp
from jax.experimental import pallas as pl


def kernel(x, edge_index, batch, params):
    raise NotImplementedError("write your pallas kernel here")


# speedup vs baseline: 6.1148x; 6.1148x over previous
import functools

import jax
import jax.numpy as jnp
from jax import lax
from jax.experimental import pallas as pl
from jax.experimental.pallas import tpu as pltpu
from jax.experimental.pallas import tpu_sc as plsc

_N = 10000
_E = 160000
_D = 256
_HALF = 128
_G = 16
_EPS = 1e-5

_NC = 2
_NS = 16
_RPT = 624
_TAIL0 = _RPT * _NS
_TAILN = _N - _TAIL0
_EPT = _E // _NS
_CHUNK = 80
_NCHUNK = _EPT // _CHUNK
_NPAD = 10240
_DRPT = _NPAD // _NS

def _deg_body(dst_h, ones_h, zeros_h, out_h, idx_v, ones_v, acc_sh):
    c = lax.axis_index("c")
    s = lax.axis_index("s")
    pltpu.sync_copy(zeros_h.at[pl.ds(s * _DRPT, _DRPT)],
                    acc_sh.at[pl.ds(s * _DRPT, _DRPT)])
    pltpu.sync_copy(ones_h, ones_v)
    plsc.subcore_barrier()
    base = s * _EPT

    def chunk(g, carry):
        pltpu.sync_copy(dst_h.at[pl.ds(base + g * _CHUNK, _CHUNK)], idx_v)
        pltpu.sync_copy(ones_v, acc_sh.at[idx_v], add=True)
        return carry

    lax.fori_loop(0, _NCHUNK, chunk, 0)
    plsc.subcore_barrier()
    pltpu.sync_copy(acc_sh.at[pl.ds(s * _DRPT, _DRPT)],
                    out_h.at[pl.ds(c * _NPAD + s * _DRPT, _DRPT)])


@functools.cache
def _sc_kernels():
    mesh = plsc.VectorSubcoreMesh(core_axis_name="c", subcore_axis_name="s",
                                  num_cores=_NC, num_subcores=_NS)
    deg = pl.kernel(
        _deg_body,
        out_type=jax.ShapeDtypeStruct((_NC * _NPAD,), jnp.float32),
        mesh=mesh,
        scratch_types=[
            pltpu.VMEM((_CHUNK,), jnp.int32),
            pltpu.VMEM((_CHUNK,), jnp.float32),
            pltpu.VMEM_SHARED((_NPAD,), jnp.float32),
        ],
    )
    conv = pl.kernel(
        _conv_body,
        out_type=jax.ShapeDtypeStruct((_NC * _N, _HALF), jnp.float32),
        mesh=mesh,
        scratch_types=[
            pltpu.VMEM((_CHUNK,), jnp.int32),
            pltpu.VMEM((_CHUNK,), jnp.int32),
            pltpu.VMEM((_CHUNK, _HALF), jnp.float32),
            pltpu.VMEM_SHARED((_N, _HALF), jnp.float32),
            pltpu.SemaphoreType.DMA,
        ],
    )
    return deg, conv


def _deg_call(*args):
    return _sc_kernels()[0](*args)


def _conv_body(p_h, srcs_h, dst_h, zeros_h, out_h, idx_s, idx_d, rows_v,
               acc_sh, sem):
    c = lax.axis_index("c")
    s = lax.axis_index("s")
    row0 = s * _RPT
    pltpu.sync_copy(zeros_h.at[pl.ds(row0, _RPT)], acc_sh.at[pl.ds(row0, _RPT)])

    @pl.when(s == _NS - 1)
    def _():
        pltpu.sync_copy(zeros_h.at[pl.ds(_TAIL0, _TAILN)],
                        acc_sh.at[pl.ds(_TAIL0, _TAILN)])

    plsc.subcore_barrier()

    def chunk(g, carry):
        off = s * _EPT + g * _CHUNK
        pltpu.sync_copy(srcs_h.at[pl.ds(c * _E + off, _CHUNK)], idx_s)
        pltpu.sync_copy(dst_h.at[pl.ds(off, _CHUNK)], idx_d)
        pltpu.async_copy(p_h.at[idx_s], rows_v, sem).wait()
        pltpu.sync_copy(rows_v, acc_sh.at[idx_d], add=True)
        return carry

    lax.fori_loop(0, _NCHUNK, chunk, 0)
    plsc.subcore_barrier()
    pltpu.sync_copy(acc_sh.at[pl.ds(row0, _RPT)],
                    out_h.at[pl.ds(c * _N + row0, _RPT)])

    @pl.when(s == _NS - 1)
    def _():
        pltpu.sync_copy(acc_sh.at[pl.ds(_TAIL0, _TAILN)],
                        out_h.at[pl.ds(c * _N + _TAIL0, _TAILN)])


def _conv_call(*args):
    return _sc_kernels()[1](*args)


_RB = 1000


def _mm_body(deg_ref, h_ref, w_ref, o_ref):
    dinv = lax.rsqrt(deg_ref[...])
    o_ref[...] = lax.dot_general(
        h_ref[...], w_ref[...], (((1,), (0,)), ((), ())),
        preferred_element_type=jnp.float32) * dinv


def _scaled_matmul(deg, h, w):
    return pl.pallas_call(
        _mm_body,
        grid=(2, _N // _RB),
        in_specs=[
            pl.BlockSpec((_RB, 1), lambda j, i: (i, 0)),
            pl.BlockSpec((_RB, _D), lambda j, i: (i, 0)),
            pl.BlockSpec((_D, _HALF), lambda j, i: (0, j)),
        ],
        out_specs=pl.BlockSpec((_RB, _HALF), lambda j, i: (j * (_N // _RB) + i, 0)),
        out_shape=jax.ShapeDtypeStruct((2 * _N, _HALF), jnp.float32),
    )(deg, h, w)


def _stats_body(acc_ref, p_ref, deg_ref, b_ref, t_ref, s1_ref, s2_ref):
    i = pl.program_id(1)
    dinv = lax.rsqrt(deg_ref[...])
    t = dinv * (acc_ref[...] + p_ref[...]) + b_ref[...]
    t_ref[...] = t

    @pl.when(i == 0)
    def _():
        s1_ref[...] = jnp.zeros_like(s1_ref)
        s2_ref[...] = jnp.zeros_like(s2_ref)

    s1_ref[...] += jnp.sum(t, axis=0, keepdims=True)
    s2_ref[...] += jnp.sum(t * t, axis=0, keepdims=True)


def _stats_call(acc, p, deg, b):
    return pl.pallas_call(
        _stats_body,
        grid=(2, _N // _RB),
        in_specs=[
            pl.BlockSpec((_RB, _HALF), lambda j, i: (j * (_N // _RB) + i, 0)),
            pl.BlockSpec((_RB, _HALF), lambda j, i: (j * (_N // _RB) + i, 0)),
            pl.BlockSpec((_RB, 1), lambda j, i: (i, 0)),
            pl.BlockSpec((1, _HALF), lambda j, i: (0, j)),
        ],
        out_specs=[
            pl.BlockSpec((_RB, _HALF), lambda j, i: (i, j)),
            pl.BlockSpec((1, _HALF), lambda j, i: (0, j)),
            pl.BlockSpec((1, _HALF), lambda j, i: (0, j)),
        ],
        out_shape=[
            jax.ShapeDtypeStruct((_N, _D), jnp.float32),
            jax.ShapeDtypeStruct((1, _D), jnp.float32),
            jax.ShapeDtypeStruct((1, _D), jnp.float32),
        ],
    )(acc, p, deg, b)


def _bn_finish(t, s1, s2, g, be):
    mean = s1 / _N
    var = s2 / _N - mean * mean
    sc = g * lax.rsqrt(var + _EPS)
    return jnp.maximum((t - mean) * sc + be, 0.0)


def _bn_body(t_ref, s1_ref, s2_ref, g_ref, be_ref, o_ref):
    o_ref[...] = _bn_finish(t_ref[...], s1_ref[...], s2_ref[...], g_ref[...],
                            be_ref[...])


def _bn_res_body(t_ref, s1_ref, s2_ref, g_ref, be_ref, id_ref, o_ref):
    o_ref[...] = _bn_finish(t_ref[...], s1_ref[...], s2_ref[...], g_ref[...],
                            be_ref[...]) + id_ref[...]


def _bn_call(t, s1, s2, g, be, identity=None):
    row_spec = pl.BlockSpec((_RB, _D), lambda i: (i, 0))
    vec_spec = pl.BlockSpec((1, _D), lambda i: (0, 0))
    if identity is None:
        body, ins, specs = _bn_body, (t, s1, s2, g, be), [row_spec] + [vec_spec] * 4
    else:
        body, ins = _bn_res_body, (t, s1, s2, g, be, identity)
        specs = [row_spec] + [vec_spec] * 4 + [row_spec]
    return pl.pallas_call(
        body,
        grid=(_N // _RB,),
        in_specs=specs,
        out_specs=row_spec,
        out_shape=jax.ShapeDtypeStruct((_N, _D), jnp.float32),
    )(*ins)


_PB = 2000


def _pool_body(batch_ref, h_ref, z_ref, ssum_ref, smax_ref, cnt_ref):
    i = pl.program_id(0)

    @pl.when(i == 0)
    def _():
        ssum_ref[...] = jnp.zeros_like(ssum_ref)
        cnt_ref[...] = jnp.zeros_like(cnt_ref)
        smax_ref[...] = jnp.full_like(smax_ref, -jnp.inf)

    b = batch_ref[...]
    h = h_ref[...]
    gids = lax.broadcasted_iota(jnp.int32, (1, _G), 1)
    onehot = (b == gids).astype(jnp.float32)
    dn = (((0,), (0,)), ((), ()))
    ssum_ref[...] += lax.dot_general(onehot, h, dn,
                                     preferred_element_type=jnp.float32, precision=lax.Precision.HIGHEST)
    cnt_ref[...] += lax.dot_general(onehot, jnp.ones_like(h), dn,
                                    preferred_element_type=jnp.float32, precision=lax.Precision.HIGHEST)
    for g in range(_G):
        cand = jnp.where(b == g, h, -jnp.inf)
        smax_ref[g:g + 1, :] = jnp.maximum(
            smax_ref[g:g + 1, :], jnp.max(cand, axis=0, keepdims=True))

    @pl.when(i == _N // _PB - 1)
    def _():
        cnt = cnt_ref[...]
        ssum = ssum_ref[...]
        mean = ssum / jnp.maximum(cnt, 1.0)
        smax = jnp.where(cnt > 0, smax_ref[...], 0.0)
        z_ref[...] = jnp.concatenate([mean, smax, ssum], axis=1)


def _pool_call(batch2d, h):
    return pl.pallas_call(
        _pool_body,
        grid=(_N // _PB,),
        in_specs=[
            pl.BlockSpec((_PB, 1), lambda i: (i, 0)),
            pl.BlockSpec((_PB, _D), lambda i: (i, 0)),
        ],
        out_specs=pl.BlockSpec((_G, 3 * _D), lambda i: (0, 0)),
        out_shape=jax.ShapeDtypeStruct((_G, 3 * _D), jnp.float32),
        scratch_shapes=[
            pltpu.VMEM((_G, _D), jnp.float32),
            pltpu.VMEM((_G, _D), jnp.float32),
            pltpu.VMEM((_G, _D), jnp.float32),
        ],
    )(batch2d, h)


def _bn_small(a, g, be):
    m = jnp.mean(a, axis=0, keepdims=True)
    v = jnp.mean((a - m) * (a - m), axis=0, keepdims=True)
    return (a - m) * lax.rsqrt(v + _EPS) * g + be


def _mlp_body(z_ref, w1_ref, b1_ref, g1_ref, be1_ref, w2_ref, b2_ref, g2_ref,
              be2_ref, w3_ref, b3_ref, o_ref):
    dn = (((1,), (0,)), ((), ()))
    a = lax.dot_general(z_ref[...], w1_ref[...], dn,
                        preferred_element_type=jnp.float32) + b1_ref[...]
    a = jnp.maximum(_bn_small(a, g1_ref[...], be1_ref[...]), 0.0)
    a = lax.dot_general(a, w2_ref[...], dn,
                        preferred_element_type=jnp.float32) + b2_ref[...]
    a = jnp.maximum(_bn_small(a, g2_ref[...], be2_ref[...]), 0.0)
    o_ref[...] = lax.dot_general(a, w3_ref[...], dn,
                                 preferred_element_type=jnp.float32) + b3_ref[...]


def _mlp_call(z, w1, b1, g1, be1, w2, b2, g2, be2, w3, b3):
    return pl.pallas_call(
        _mlp_body,
        out_shape=jax.ShapeDtypeStruct((_G, 1), jnp.float32),
    )(z, w1, b1, g1, be1, w2, b2, g2, be2, w3, b3)


def kernel(x, edge_index, batch, params):
    src = edge_index[0]
    dst = edge_index[1]
    srcs = jnp.concatenate([src, src + _N])
    zeros1 = jnp.zeros((_NPAD,), jnp.float32)
    zeros2 = jnp.zeros((_N, _HALF), jnp.float32)
    ones_c = jnp.ones((_CHUNK,), jnp.float32)

    degp = _deg_call(dst, ones_c, zeros1)
    deg = (1.0 + degp[:_N]).reshape(_N, 1)

    h = x
    identity = None
    for li in range(1, 5):
        w = params[f'W{li}']
        b = params[f'b{li}'].reshape(1, _D)
        g = params[f'g{li}'].reshape(1, _D)
        be = params[f'be{li}'].reshape(1, _D)
        p = _scaled_matmul(deg, h, w)
        acc = _conv_call(p, srcs, dst, zeros2)
        t, s1, s2 = _stats_call(acc, p, deg, b)
        h = _bn_call(t, s1, s2, g, be, identity)
        identity = h

    z = _pool_call(batch.reshape(_N, 1), h)
    pp = params
    return _mlp_call(
        z, pp['Wf1'], pp['bf1'].reshape(1, 2 * _D), pp['gf1'].reshape(1, 2 * _D),
        pp['bef1'].reshape(1, 2 * _D), pp['Wf2'], pp['bf2'].reshape(1, _D),
        pp['gf2'].reshape(1, _D), pp['bef2'].reshape(1, _D), pp['Wf3'],
        pp['bf3'].reshape(1, 1))


# speedup vs baseline: 8.9121x; 1.4575x over previous
import functools

import jax
import jax.numpy as jnp
from jax import lax
from jax.experimental import pallas as pl
from jax.experimental.pallas import tpu as pltpu
from jax.experimental.pallas import tpu_sc as plsc

_N = 10000
_E = 160000
_D = 256
_HALF = 128
_G = 16
_EPS = 1e-5

_NC = 2
_NS = 16
_RPT = 624
_TAIL0 = _RPT * _NS
_TAILN = _N - _TAIL0
_EPT = _E // _NS
_CHUNK = 80
_NCHUNK = _EPT // _CHUNK
_NPAD = 10240
_DRPT = _NPAD // _NS

def _deg_body(dst_h, ones_h, zeros_h, out_h, idx_v, ones_v, acc_sh):
    c = lax.axis_index("c")
    s = lax.axis_index("s")
    pltpu.sync_copy(zeros_h.at[pl.ds(s * _DRPT, _DRPT)],
                    acc_sh.at[pl.ds(s * _DRPT, _DRPT)])
    pltpu.sync_copy(ones_h, ones_v)
    plsc.subcore_barrier()
    base = s * _EPT

    def chunk(g, carry):
        pltpu.sync_copy(dst_h.at[pl.ds(base + g * _CHUNK, _CHUNK)], idx_v)
        pltpu.sync_copy(ones_v, acc_sh.at[idx_v], add=True)
        return carry

    lax.fori_loop(0, _NCHUNK, chunk, 0)
    plsc.subcore_barrier()
    pltpu.sync_copy(acc_sh.at[pl.ds(s * _DRPT, _DRPT)],
                    out_h.at[pl.ds(c * _NPAD + s * _DRPT, _DRPT)])


@functools.cache
def _sc_kernels():
    mesh = plsc.VectorSubcoreMesh(core_axis_name="c", subcore_axis_name="s",
                                  num_cores=_NC, num_subcores=_NS)
    deg = pl.kernel(
        _deg_body,
        out_type=jax.ShapeDtypeStruct((_NC * _NPAD,), jnp.float32),
        mesh=mesh,
        scratch_types=[
            pltpu.VMEM((_CHUNK,), jnp.int32),
            pltpu.VMEM((_CHUNK,), jnp.float32),
            pltpu.VMEM_SHARED((_NPAD,), jnp.float32),
        ],
    )
    conv = pl.kernel(
        _conv_body,
        out_type=jax.ShapeDtypeStruct((_NC * _N, _HALF), jnp.float32),
        mesh=mesh,
        scratch_types=[
            pltpu.VMEM((_CHUNK,), jnp.int32),
            pltpu.VMEM((_CHUNK,), jnp.int32),
            pltpu.VMEM((_CHUNK, _HALF), jnp.float32),
            pltpu.VMEM((_CHUNK,), jnp.int32),
            pltpu.VMEM((_CHUNK,), jnp.int32),
            pltpu.VMEM((_CHUNK, _HALF), jnp.float32),
            pltpu.VMEM_SHARED((_N, _HALF), jnp.float32),
            pltpu.SemaphoreType.DMA,
            pltpu.SemaphoreType.DMA,
        ],
    )
    return deg, conv


def _deg_call(*args):
    return _sc_kernels()[0](*args)


def _conv_body(p_h, srcs_h, dst_h, zeros_h, out_h, idx_sa, idx_da, rows_a,
               idx_sb, idx_db, rows_b, acc_sh, sem_a, sem_b):
    c = lax.axis_index("c")
    s = lax.axis_index("s")
    row0 = s * _RPT
    pltpu.sync_copy(zeros_h.at[pl.ds(row0, _RPT)], acc_sh.at[pl.ds(row0, _RPT)])

    @pl.when(s == _NS - 1)
    def _():
        pltpu.sync_copy(zeros_h.at[pl.ds(_TAIL0, _TAILN)],
                        acc_sh.at[pl.ds(_TAIL0, _TAILN)])

    plsc.subcore_barrier()

    base = s * _EPT
    pltpu.sync_copy(srcs_h.at[pl.ds(c * _E + base, _CHUNK)], idx_sa)
    pltpu.sync_copy(dst_h.at[pl.ds(base, _CHUNK)], idx_da)
    pltpu.async_copy(p_h.at[idx_sa], rows_a, sem_a)

    def chunk(g, carry):
        off = base + g * _CHUNK

        @pl.when(g % 2 == 0)
        def _():
            @pl.when(g + 1 < _NCHUNK)
            def _():
                pltpu.sync_copy(srcs_h.at[pl.ds(c * _E + off + _CHUNK, _CHUNK)],
                                idx_sb)
                pltpu.sync_copy(dst_h.at[pl.ds(off + _CHUNK, _CHUNK)], idx_db)
                pltpu.async_copy(p_h.at[idx_sb], rows_b, sem_b)

            pltpu.make_async_copy(p_h.at[idx_sa], rows_a, sem_a).wait()
            pltpu.sync_copy(rows_a, acc_sh.at[idx_da], add=True)

        @pl.when(g % 2 == 1)
        def _():
            @pl.when(g + 1 < _NCHUNK)
            def _():
                pltpu.sync_copy(srcs_h.at[pl.ds(c * _E + off + _CHUNK, _CHUNK)],
                                idx_sa)
                pltpu.sync_copy(dst_h.at[pl.ds(off + _CHUNK, _CHUNK)], idx_da)
                pltpu.async_copy(p_h.at[idx_sa], rows_a, sem_a)

            pltpu.make_async_copy(p_h.at[idx_sb], rows_b, sem_b).wait()
            pltpu.sync_copy(rows_b, acc_sh.at[idx_db], add=True)

        return carry

    lax.fori_loop(0, _NCHUNK, chunk, 0)
    plsc.subcore_barrier()
    pltpu.sync_copy(acc_sh.at[pl.ds(row0, _RPT)],
                    out_h.at[pl.ds(c * _N + row0, _RPT)])

    @pl.when(s == _NS - 1)
    def _():
        pltpu.sync_copy(acc_sh.at[pl.ds(_TAIL0, _TAILN)],
                        out_h.at[pl.ds(c * _N + _TAIL0, _TAILN)])


def _conv_call(*args):
    return _sc_kernels()[1](*args)


_RB = 1000


def _mm_body(deg_ref, h_ref, w_ref, o_ref):
    dinv = lax.rsqrt(deg_ref[...])
    o_ref[...] = lax.dot_general(
        h_ref[...], w_ref[...], (((1,), (0,)), ((), ())),
        preferred_element_type=jnp.float32) * dinv


def _scaled_matmul(deg, h, w):
    return pl.pallas_call(
        _mm_body,
        grid=(2, _N // _RB),
        in_specs=[
            pl.BlockSpec((_RB, 1), lambda j, i: (i, 0)),
            pl.BlockSpec((_RB, _D), lambda j, i: (i, 0)),
            pl.BlockSpec((_D, _HALF), lambda j, i: (0, j)),
        ],
        out_specs=pl.BlockSpec((_RB, _HALF), lambda j, i: (j * (_N // _RB) + i, 0)),
        out_shape=jax.ShapeDtypeStruct((2 * _N, _HALF), jnp.float32),
    )(deg, h, w)


def _stats_body(acc_ref, p_ref, deg_ref, b_ref, t_ref, s1_ref, s2_ref):
    i = pl.program_id(1)
    dinv = lax.rsqrt(deg_ref[...])
    t = dinv * (acc_ref[...] + p_ref[...]) + b_ref[...]
    t_ref[...] = t

    @pl.when(i == 0)
    def _():
        s1_ref[...] = jnp.zeros_like(s1_ref)
        s2_ref[...] = jnp.zeros_like(s2_ref)

    s1_ref[...] += jnp.sum(t, axis=0, keepdims=True)
    s2_ref[...] += jnp.sum(t * t, axis=0, keepdims=True)


def _stats_call(acc, p, deg, b):
    return pl.pallas_call(
        _stats_body,
        grid=(2, _N // _RB),
        in_specs=[
            pl.BlockSpec((_RB, _HALF), lambda j, i: (j * (_N // _RB) + i, 0)),
            pl.BlockSpec((_RB, _HALF), lambda j, i: (j * (_N // _RB) + i, 0)),
            pl.BlockSpec((_RB, 1), lambda j, i: (i, 0)),
            pl.BlockSpec((1, _HALF), lambda j, i: (0, j)),
        ],
        out_specs=[
            pl.BlockSpec((_RB, _HALF), lambda j, i: (i, j)),
            pl.BlockSpec((1, _HALF), lambda j, i: (0, j)),
            pl.BlockSpec((1, _HALF), lambda j, i: (0, j)),
        ],
        out_shape=[
            jax.ShapeDtypeStruct((_N, _D), jnp.float32),
            jax.ShapeDtypeStruct((1, _D), jnp.float32),
            jax.ShapeDtypeStruct((1, _D), jnp.float32),
        ],
    )(acc, p, deg, b)


def _bn_finish(t, s1, s2, g, be):
    mean = s1 / _N
    var = s2 / _N - mean * mean
    sc = g * lax.rsqrt(var + _EPS)
    return jnp.maximum((t - mean) * sc + be, 0.0)


def _bn_body(t_ref, s1_ref, s2_ref, g_ref, be_ref, o_ref):
    o_ref[...] = _bn_finish(t_ref[...], s1_ref[...], s2_ref[...], g_ref[...],
                            be_ref[...])


def _bn_res_body(t_ref, s1_ref, s2_ref, g_ref, be_ref, id_ref, o_ref):
    o_ref[...] = _bn_finish(t_ref[...], s1_ref[...], s2_ref[...], g_ref[...],
                            be_ref[...]) + id_ref[...]


def _bn_call(t, s1, s2, g, be, identity=None):
    row_spec = pl.BlockSpec((_RB, _D), lambda i: (i, 0))
    vec_spec = pl.BlockSpec((1, _D), lambda i: (0, 0))
    if identity is None:
        body, ins, specs = _bn_body, (t, s1, s2, g, be), [row_spec] + [vec_spec] * 4
    else:
        body, ins = _bn_res_body, (t, s1, s2, g, be, identity)
        specs = [row_spec] + [vec_spec] * 4 + [row_spec]
    return pl.pallas_call(
        body,
        grid=(_N // _RB,),
        in_specs=specs,
        out_specs=row_spec,
        out_shape=jax.ShapeDtypeStruct((_N, _D), jnp.float32),
    )(*ins)


_PB = 2000


def _pool_body(batch_ref, h_ref, z_ref, ssum_ref, smax_ref, cnt_ref):
    i = pl.program_id(0)

    @pl.when(i == 0)
    def _():
        ssum_ref[...] = jnp.zeros_like(ssum_ref)
        cnt_ref[...] = jnp.zeros_like(cnt_ref)
        smax_ref[...] = jnp.full_like(smax_ref, -jnp.inf)

    b = batch_ref[...]
    h = h_ref[...]
    gids = lax.broadcasted_iota(jnp.int32, (1, _G), 1)
    onehot = (b == gids).astype(jnp.float32)
    dn = (((0,), (0,)), ((), ()))
    ssum_ref[...] += lax.dot_general(onehot, h, dn,
                                     preferred_element_type=jnp.float32, precision=lax.Precision.HIGHEST)
    cnt_ref[...] += lax.dot_general(onehot, jnp.ones_like(h), dn,
                                    preferred_element_type=jnp.float32, precision=lax.Precision.HIGHEST)
    for g in range(_G):
        cand = jnp.where(b == g, h, -jnp.inf)
        smax_ref[g:g + 1, :] = jnp.maximum(
            smax_ref[g:g + 1, :], jnp.max(cand, axis=0, keepdims=True))

    @pl.when(i == _N // _PB - 1)
    def _():
        cnt = cnt_ref[...]
        ssum = ssum_ref[...]
        mean = ssum / jnp.maximum(cnt, 1.0)
        smax = jnp.where(cnt > 0, smax_ref[...], 0.0)
        z_ref[...] = jnp.concatenate([mean, smax, ssum], axis=1)


def _pool_call(batch2d, h):
    return pl.pallas_call(
        _pool_body,
        grid=(_N // _PB,),
        in_specs=[
            pl.BlockSpec((_PB, 1), lambda i: (i, 0)),
            pl.BlockSpec((_PB, _D), lambda i: (i, 0)),
        ],
        out_specs=pl.BlockSpec((_G, 3 * _D), lambda i: (0, 0)),
        out_shape=jax.ShapeDtypeStruct((_G, 3 * _D), jnp.float32),
        scratch_shapes=[
            pltpu.VMEM((_G, _D), jnp.float32),
            pltpu.VMEM((_G, _D), jnp.float32),
            pltpu.VMEM((_G, _D), jnp.float32),
        ],
    )(batch2d, h)


def _bn_small(a, g, be):
    m = jnp.mean(a, axis=0, keepdims=True)
    v = jnp.mean((a - m) * (a - m), axis=0, keepdims=True)
    return (a - m) * lax.rsqrt(v + _EPS) * g + be


def _mlp_body(z_ref, w1_ref, b1_ref, g1_ref, be1_ref, w2_ref, b2_ref, g2_ref,
              be2_ref, w3_ref, b3_ref, o_ref):
    dn = (((1,), (0,)), ((), ()))
    a = lax.dot_general(z_ref[...], w1_ref[...], dn,
                        preferred_element_type=jnp.float32) + b1_ref[...]
    a = jnp.maximum(_bn_small(a, g1_ref[...], be1_ref[...]), 0.0)
    a = lax.dot_general(a, w2_ref[...], dn,
                        preferred_element_type=jnp.float32) + b2_ref[...]
    a = jnp.maximum(_bn_small(a, g2_ref[...], be2_ref[...]), 0.0)
    o_ref[...] = lax.dot_general(a, w3_ref[...], dn,
                                 preferred_element_type=jnp.float32) + b3_ref[...]


def _mlp_call(z, w1, b1, g1, be1, w2, b2, g2, be2, w3, b3):
    return pl.pallas_call(
        _mlp_body,
        out_shape=jax.ShapeDtypeStruct((_G, 1), jnp.float32),
    )(z, w1, b1, g1, be1, w2, b2, g2, be2, w3, b3)


def kernel(x, edge_index, batch, params):
    src = edge_index[0]
    dst = edge_index[1]
    srcs = jnp.concatenate([src, src + _N])
    zeros1 = jnp.zeros((_NPAD,), jnp.float32)
    zeros2 = jnp.zeros((_N, _HALF), jnp.float32)
    ones_c = jnp.ones((_CHUNK,), jnp.float32)

    degp = _deg_call(dst, ones_c, zeros1)
    deg = (1.0 + degp[:_N]).reshape(_N, 1)

    h = x
    identity = None
    for li in range(1, 5):
        w = params[f'W{li}']
        b = params[f'b{li}'].reshape(1, _D)
        g = params[f'g{li}'].reshape(1, _D)
        be = params[f'be{li}'].reshape(1, _D)
        p = _scaled_matmul(deg, h, w)
        acc = _conv_call(p, srcs, dst, zeros2)
        t, s1, s2 = _stats_call(acc, p, deg, b)
        h = _bn_call(t, s1, s2, g, be, identity)
        identity = h

    z = _pool_call(batch.reshape(_N, 1), h)
    pp = params
    return _mlp_call(
        z, pp['Wf1'], pp['bf1'].reshape(1, 2 * _D), pp['gf1'].reshape(1, 2 * _D),
        pp['bef1'].reshape(1, 2 * _D), pp['Wf2'], pp['bf2'].reshape(1, _D),
        pp['gf2'].reshape(1, _D), pp['bef2'].reshape(1, _D), pp['Wf3'],
        pp['bf3'].reshape(1, 1))


# speedup vs baseline: 10.3445x; 1.1607x over previous
import functools

import jax
import jax.numpy as jnp
from jax import lax
from jax.experimental import pallas as pl
from jax.experimental.pallas import tpu as pltpu
from jax.experimental.pallas import tpu_sc as plsc

_N = 10000
_E = 160000
_D = 256
_HALF = 128
_G = 16
_EPS = 1e-5

_NC = 2
_NS = 16
_RPT = 624
_TAIL0 = _RPT * _NS
_TAILN = _N - _TAIL0
_EPT = _E // _NS
_CHUNK = 128
_NCHUNK = _EPT // _CHUNK
_TEDGE = _EPT - _NCHUNK * _CHUNK
_TOFF = _NCHUNK * _CHUNK
_NPAD = 10240
_DRPT = _NPAD // _NS

def _deg_body(dst_h, ones_h, zeros_h, out_h, idx_v, ones_v, idx_t, ones_t,
              acc_sh):
    c = lax.axis_index("c")
    s = lax.axis_index("s")
    pltpu.sync_copy(zeros_h.at[pl.ds(s * _DRPT, _DRPT)],
                    acc_sh.at[pl.ds(s * _DRPT, _DRPT)])
    pltpu.sync_copy(ones_h, ones_v)
    plsc.subcore_barrier()
    base = s * _EPT

    def chunk(g, carry):
        pltpu.sync_copy(dst_h.at[pl.ds(base + g * _CHUNK, _CHUNK)], idx_v)
        pltpu.sync_copy(ones_v, acc_sh.at[idx_v], add=True)
        return carry

    lax.fori_loop(0, _NCHUNK, chunk, 0)
    pltpu.sync_copy(dst_h.at[pl.ds(base + _TOFF, _TEDGE)], idx_t)
    pltpu.sync_copy(ones_h.at[pl.ds(0, _TEDGE)], ones_t)
    pltpu.sync_copy(ones_t, acc_sh.at[idx_t], add=True)
    plsc.subcore_barrier()
    pltpu.sync_copy(acc_sh.at[pl.ds(s * _DRPT, _DRPT)],
                    out_h.at[pl.ds(c * _NPAD + s * _DRPT, _DRPT)])


@functools.cache
def _sc_kernels():
    mesh = plsc.VectorSubcoreMesh(core_axis_name="c", subcore_axis_name="s",
                                  num_cores=_NC, num_subcores=_NS)
    deg = pl.kernel(
        _deg_body,
        out_type=jax.ShapeDtypeStruct((_NC * _NPAD,), jnp.float32),
        mesh=mesh,
        scratch_types=[
            pltpu.VMEM((_CHUNK,), jnp.int32),
            pltpu.VMEM((_CHUNK,), jnp.float32),
            pltpu.VMEM((_TEDGE,), jnp.int32),
            pltpu.VMEM((_TEDGE,), jnp.float32),
            pltpu.VMEM_SHARED((_NPAD,), jnp.float32),
        ],
    )
    conv = pl.kernel(
        _conv_body,
        out_type=jax.ShapeDtypeStruct((_NC * _N, _HALF), jnp.float32),
        mesh=mesh,
        scratch_types=[
            pltpu.VMEM((_CHUNK,), jnp.int32),
            pltpu.VMEM((_CHUNK,), jnp.int32),
            pltpu.VMEM((_CHUNK, _HALF), jnp.float32),
            pltpu.VMEM((_CHUNK,), jnp.int32),
            pltpu.VMEM((_CHUNK,), jnp.int32),
            pltpu.VMEM((_CHUNK, _HALF), jnp.float32),
            pltpu.VMEM((_TEDGE,), jnp.int32),
            pltpu.VMEM((_TEDGE,), jnp.int32),
            pltpu.VMEM((_TEDGE, _HALF), jnp.float32),
            pltpu.VMEM_SHARED((_N, _HALF), jnp.float32),
            pltpu.SemaphoreType.DMA,
            pltpu.SemaphoreType.DMA,
        ],
    )
    return deg, conv


def _deg_call(*args):
    return _sc_kernels()[0](*args)


def _conv_body(p_h, srcs_h, dst_h, zeros_h, out_h, idx_sa, idx_da, rows_a,
               idx_sb, idx_db, rows_b, idx_st, idx_dt, rows_t, acc_sh,
               sem_a, sem_b):
    c = lax.axis_index("c")
    s = lax.axis_index("s")
    row0 = s * _RPT
    pltpu.sync_copy(zeros_h.at[pl.ds(row0, _RPT)], acc_sh.at[pl.ds(row0, _RPT)])

    @pl.when(s == _NS - 1)
    def _():
        pltpu.sync_copy(zeros_h.at[pl.ds(_TAIL0, _TAILN)],
                        acc_sh.at[pl.ds(_TAIL0, _TAILN)])

    plsc.subcore_barrier()

    base = s * _EPT
    pltpu.sync_copy(srcs_h.at[pl.ds(c * _E + base, _CHUNK)], idx_sa)
    pltpu.sync_copy(dst_h.at[pl.ds(base, _CHUNK)], idx_da)
    pltpu.async_copy(p_h.at[idx_sa], rows_a, sem_a)

    def chunk(g, carry):
        off = base + g * _CHUNK

        @pl.when(g % 2 == 0)
        def _():
            @pl.when(g + 1 < _NCHUNK)
            def _():
                pltpu.sync_copy(srcs_h.at[pl.ds(c * _E + off + _CHUNK, _CHUNK)],
                                idx_sb)
                pltpu.sync_copy(dst_h.at[pl.ds(off + _CHUNK, _CHUNK)], idx_db)
                pltpu.async_copy(p_h.at[idx_sb], rows_b, sem_b)

            pltpu.make_async_copy(p_h.at[idx_sa], rows_a, sem_a).wait()
            pltpu.sync_copy(rows_a, acc_sh.at[idx_da], add=True)

        @pl.when(g % 2 == 1)
        def _():
            @pl.when(g + 1 < _NCHUNK)
            def _():
                pltpu.sync_copy(srcs_h.at[pl.ds(c * _E + off + _CHUNK, _CHUNK)],
                                idx_sa)
                pltpu.sync_copy(dst_h.at[pl.ds(off + _CHUNK, _CHUNK)], idx_da)
                pltpu.async_copy(p_h.at[idx_sa], rows_a, sem_a)

            pltpu.make_async_copy(p_h.at[idx_sb], rows_b, sem_b).wait()
            pltpu.sync_copy(rows_b, acc_sh.at[idx_db], add=True)

        return carry

    lax.fori_loop(0, _NCHUNK, chunk, 0)
    pltpu.sync_copy(srcs_h.at[pl.ds(c * _E + base + _TOFF, _TEDGE)], idx_st)
    pltpu.sync_copy(dst_h.at[pl.ds(base + _TOFF, _TEDGE)], idx_dt)
    pltpu.async_copy(p_h.at[idx_st], rows_t, sem_a).wait()
    pltpu.sync_copy(rows_t, acc_sh.at[idx_dt], add=True)
    plsc.subcore_barrier()
    pltpu.sync_copy(acc_sh.at[pl.ds(row0, _RPT)],
                    out_h.at[pl.ds(c * _N + row0, _RPT)])

    @pl.when(s == _NS - 1)
    def _():
        pltpu.sync_copy(acc_sh.at[pl.ds(_TAIL0, _TAILN)],
                        out_h.at[pl.ds(c * _N + _TAIL0, _TAILN)])


def _conv_call(*args):
    return _sc_kernels()[1](*args)


_RB = 1000


def _mm_body(deg_ref, h_ref, w_ref, o_ref):
    dinv = lax.rsqrt(deg_ref[...])
    o_ref[...] = lax.dot_general(
        h_ref[...], w_ref[...], (((1,), (0,)), ((), ())),
        preferred_element_type=jnp.float32) * dinv


def _scaled_matmul(deg, h, w):
    return pl.pallas_call(
        _mm_body,
        grid=(2, _N // _RB),
        in_specs=[
            pl.BlockSpec((_RB, 1), lambda j, i: (i, 0)),
            pl.BlockSpec((_RB, _D), lambda j, i: (i, 0)),
            pl.BlockSpec((_D, _HALF), lambda j, i: (0, j)),
        ],
        out_specs=pl.BlockSpec((_RB, _HALF), lambda j, i: (j * (_N // _RB) + i, 0)),
        out_shape=jax.ShapeDtypeStruct((2 * _N, _HALF), jnp.float32),
    )(deg, h, w)


def _stats_body(acc_ref, p_ref, deg_ref, b_ref, t_ref, s1_ref, s2_ref):
    i = pl.program_id(1)
    dinv = lax.rsqrt(deg_ref[...])
    t = dinv * (acc_ref[...] + p_ref[...]) + b_ref[...]
    t_ref[...] = t

    @pl.when(i == 0)
    def _():
        s1_ref[...] = jnp.zeros_like(s1_ref)
        s2_ref[...] = jnp.zeros_like(s2_ref)

    s1_ref[...] += jnp.sum(t, axis=0, keepdims=True)
    s2_ref[...] += jnp.sum(t * t, axis=0, keepdims=True)


def _stats_call(acc, p, deg, b):
    return pl.pallas_call(
        _stats_body,
        grid=(2, _N // _RB),
        in_specs=[
            pl.BlockSpec((_RB, _HALF), lambda j, i: (j * (_N // _RB) + i, 0)),
            pl.BlockSpec((_RB, _HALF), lambda j, i: (j * (_N // _RB) + i, 0)),
            pl.BlockSpec((_RB, 1), lambda j, i: (i, 0)),
            pl.BlockSpec((1, _HALF), lambda j, i: (0, j)),
        ],
        out_specs=[
            pl.BlockSpec((_RB, _HALF), lambda j, i: (i, j)),
            pl.BlockSpec((1, _HALF), lambda j, i: (0, j)),
            pl.BlockSpec((1, _HALF), lambda j, i: (0, j)),
        ],
        out_shape=[
            jax.ShapeDtypeStruct((_N, _D), jnp.float32),
            jax.ShapeDtypeStruct((1, _D), jnp.float32),
            jax.ShapeDtypeStruct((1, _D), jnp.float32),
        ],
    )(acc, p, deg, b)


def _bn_finish(t, s1, s2, g, be):
    mean = s1 / _N
    var = s2 / _N - mean * mean
    sc = g * lax.rsqrt(var + _EPS)
    return jnp.maximum((t - mean) * sc + be, 0.0)


def _bn_body(t_ref, s1_ref, s2_ref, g_ref, be_ref, o_ref):
    o_ref[...] = _bn_finish(t_ref[...], s1_ref[...], s2_ref[...], g_ref[...],
                            be_ref[...])


def _bn_res_body(t_ref, s1_ref, s2_ref, g_ref, be_ref, id_ref, o_ref):
    o_ref[...] = _bn_finish(t_ref[...], s1_ref[...], s2_ref[...], g_ref[...],
                            be_ref[...]) + id_ref[...]


def _bn_call(t, s1, s2, g, be, identity=None):
    row_spec = pl.BlockSpec((_RB, _D), lambda i: (i, 0))
    vec_spec = pl.BlockSpec((1, _D), lambda i: (0, 0))
    if identity is None:
        body, ins, specs = _bn_body, (t, s1, s2, g, be), [row_spec] + [vec_spec] * 4
    else:
        body, ins = _bn_res_body, (t, s1, s2, g, be, identity)
        specs = [row_spec] + [vec_spec] * 4 + [row_spec]
    return pl.pallas_call(
        body,
        grid=(_N // _RB,),
        in_specs=specs,
        out_specs=row_spec,
        out_shape=jax.ShapeDtypeStruct((_N, _D), jnp.float32),
    )(*ins)


_PB = 2000


def _pool_body(batch_ref, h_ref, z_ref, ssum_ref, smax_ref, cnt_ref):
    i = pl.program_id(0)

    @pl.when(i == 0)
    def _():
        ssum_ref[...] = jnp.zeros_like(ssum_ref)
        cnt_ref[...] = jnp.zeros_like(cnt_ref)
        smax_ref[...] = jnp.full_like(smax_ref, -jnp.inf)

    b = batch_ref[...]
    h = h_ref[...]
    gids = lax.broadcasted_iota(jnp.int32, (1, _G), 1)
    onehot = (b == gids).astype(jnp.float32)
    dn = (((0,), (0,)), ((), ()))
    ssum_ref[...] += lax.dot_general(onehot, h, dn,
                                     preferred_element_type=jnp.float32, precision=lax.Precision.HIGHEST)
    cnt_ref[...] += lax.dot_general(onehot, jnp.ones_like(h), dn,
                                    preferred_element_type=jnp.float32, precision=lax.Precision.HIGHEST)
    for g in range(_G):
        cand = jnp.where(b == g, h, -jnp.inf)
        smax_ref[g:g + 1, :] = jnp.maximum(
            smax_ref[g:g + 1, :], jnp.max(cand, axis=0, keepdims=True))

    @pl.when(i == _N // _PB - 1)
    def _():
        cnt = cnt_ref[...]
        ssum = ssum_ref[...]
        mean = ssum / jnp.maximum(cnt, 1.0)
        smax = jnp.where(cnt > 0, smax_ref[...], 0.0)
        z_ref[...] = jnp.concatenate([mean, smax, ssum], axis=1)


def _pool_call(batch2d, h):
    return pl.pallas_call(
        _pool_body,
        grid=(_N // _PB,),
        in_specs=[
            pl.BlockSpec((_PB, 1), lambda i: (i, 0)),
            pl.BlockSpec((_PB, _D), lambda i: (i, 0)),
        ],
        out_specs=pl.BlockSpec((_G, 3 * _D), lambda i: (0, 0)),
        out_shape=jax.ShapeDtypeStruct((_G, 3 * _D), jnp.float32),
        scratch_shapes=[
            pltpu.VMEM((_G, _D), jnp.float32),
            pltpu.VMEM((_G, _D), jnp.float32),
            pltpu.VMEM((_G, _D), jnp.float32),
        ],
    )(batch2d, h)


def _bn_small(a, g, be):
    m = jnp.mean(a, axis=0, keepdims=True)
    v = jnp.mean((a - m) * (a - m), axis=0, keepdims=True)
    return (a - m) * lax.rsqrt(v + _EPS) * g + be


def _mlp_body(z_ref, w1_ref, b1_ref, g1_ref, be1_ref, w2_ref, b2_ref, g2_ref,
              be2_ref, w3_ref, b3_ref, o_ref):
    dn = (((1,), (0,)), ((), ()))
    a = lax.dot_general(z_ref[...], w1_ref[...], dn,
                        preferred_element_type=jnp.float32) + b1_ref[...]
    a = jnp.maximum(_bn_small(a, g1_ref[...], be1_ref[...]), 0.0)
    a = lax.dot_general(a, w2_ref[...], dn,
                        preferred_element_type=jnp.float32) + b2_ref[...]
    a = jnp.maximum(_bn_small(a, g2_ref[...], be2_ref[...]), 0.0)
    o_ref[...] = lax.dot_general(a, w3_ref[...], dn,
                                 preferred_element_type=jnp.float32) + b3_ref[...]


def _mlp_call(z, w1, b1, g1, be1, w2, b2, g2, be2, w3, b3):
    return pl.pallas_call(
        _mlp_body,
        out_shape=jax.ShapeDtypeStruct((_G, 1), jnp.float32),
    )(z, w1, b1, g1, be1, w2, b2, g2, be2, w3, b3)


def kernel(x, edge_index, batch, params):
    src = edge_index[0]
    dst = edge_index[1]
    srcs = jnp.concatenate([src, src + _N])
    zeros1 = jnp.zeros((_NPAD,), jnp.float32)
    zeros2 = jnp.zeros((_N, _HALF), jnp.float32)
    ones_c = jnp.ones((_CHUNK,), jnp.float32)

    degp = _deg_call(dst, ones_c, zeros1)
    deg = (1.0 + degp[:_N]).reshape(_N, 1)

    h = x
    identity = None
    for li in range(1, 5):
        w = params[f'W{li}']
        b = params[f'b{li}'].reshape(1, _D)
        g = params[f'g{li}'].reshape(1, _D)
        be = params[f'be{li}'].reshape(1, _D)
        p = _scaled_matmul(deg, h, w)
        acc = _conv_call(p, srcs, dst, zeros2)
        t, s1, s2 = _stats_call(acc, p, deg, b)
        h = _bn_call(t, s1, s2, g, be, identity)
        identity = h

    z = _pool_call(batch.reshape(_N, 1), h)
    pp = params
    return _mlp_call(
        z, pp['Wf1'], pp['bf1'].reshape(1, 2 * _D), pp['gf1'].reshape(1, 2 * _D),
        pp['bef1'].reshape(1, 2 * _D), pp['Wf2'], pp['bf2'].reshape(1, _D),
        pp['gf2'].reshape(1, _D), pp['bef2'].reshape(1, _D), pp['Wf3'],
        pp['bf3'].reshape(1, 1))
